# Initial kernel scaffold; baseline (speedup 1.0000x reference)
import jax
import jax.numpy as jnp
from jax.experimental import pallas as pl


def kernel(nodes, emb, neigh_idx, W):
    raise NotImplementedError("write your pallas kernel here")


# speedup vs baseline: 2.5080x; 2.5080x over previous
import functools

import jax
import jax.numpy as jnp
from jax import lax
from jax.experimental import pallas as pl
from jax.experimental.pallas import tpu as pltpu
from jax.experimental.pallas import tpu_sc as plsc

_D = 256
_K = 16
_NC = 2
_NS = 16
_NW = _NC * _NS
_BP = 10240
_BW = _BP // _NW
_CH = 8
_NCHUNK = _BW // _CH


def _sc_body(nodes_hbm, nidx_hbm, emb_hbm, self_hbm, neigh_hbm,
             nodes_v, nids2_v, nidsf_v, rows0_v, rows1_v, acc_sp, didx0_v,
             didx1_v, zbuf_v, selfbuf_v, sem_n, sem_s, sem_g0, sem_g1,
             sem_o0, sem_o1):
    c = lax.axis_index("c")
    s = lax.axis_index("s")
    wid = c * _NS + s
    base = wid * _BW
    sbase = s * (2 * _CH)

    pltpu.sync_copy(nodes_hbm.at[pl.ds(base, _BW)], nodes_v)

    cps = []
    for k in range(4):
        cps.append(pltpu.async_copy(
            nidx_hbm.at[nodes_v.at[pl.ds(k * 80, 80)]],
            nids2_v.at[pl.ds(k * 80, 80)], sem_n))
    for cp in cps:
        cp.wait()

    def _flat(i, _):
        nidsf_v[pl.ds(i * _K, _K)] = nids2_v[i, :]
        return 0
    lax.fori_loop(0, _BW, _flat, 0)

    for k in range(4):
        pltpu.async_copy(
            emb_hbm.at[nodes_v.at[pl.ds(k * 80, 80)]], selfbuf_v, sem_s
        ).wait()
        pltpu.sync_copy(selfbuf_v, self_hbm.at[pl.ds(base + k * 80, 80)])

    zero16 = jnp.zeros((16,), jnp.float32)
    for r in range(_CH):
        for g in range(_D // 16):
            zbuf_v[r, pl.ds(g * 16, 16)] = zero16
    ones16 = jnp.full((_K,), 1, jnp.int32)
    for g in range(_CH):
        didx0_v[pl.ds(g * _K, _K)] = ones16 * (sbase + g)
        didx1_v[pl.ds(g * _K, _K)] = ones16 * (sbase + _CH + g)

    rows = (rows0_v, rows1_v)
    sems = (sem_g0, sem_g1)
    didxs = (didx0_v, didx1_v)
    sems_o = (sem_o0, sem_o1)

    def _fire(ci, b):
        return pltpu.async_copy(
            emb_hbm.at[nidsf_v.at[pl.ds(ci * (_CH * _K), _CH * _K)]],
            rows[b], sems[b])

    _fire(0, 0)
    _fire(1, 1)

    def _pair(p, _):
        for b in range(2):
            ci = p * 2 + b
            slot = acc_sp.at[pl.ds(sbase + b * _CH, _CH)]
            @pl.when(p > 0)
            def _():
                pltpu.make_async_copy(
                    slot, neigh_hbm.at[pl.ds(base, _CH)], sems_o[b]).wait()
            pltpu.sync_copy(zbuf_v, slot)
            pltpu.make_async_copy(
                emb_hbm.at[nidsf_v.at[pl.ds(ci * (_CH * _K), _CH * _K)]],
                rows[b], sems[b]).wait()
            pltpu.sync_copy(rows[b], acc_sp.at[didxs[b]], add=True)
            pltpu.async_copy(
                slot, neigh_hbm.at[pl.ds(base + ci * _CH, _CH)], sems_o[b])
            @pl.when(ci + 2 < _NCHUNK)
            def _():
                _fire(ci + 2, b)
        return 0
    lax.fori_loop(0, _NCHUNK // 2, _pair, 0)

    for b in range(2):
        pltpu.make_async_copy(
            acc_sp.at[pl.ds(sbase + b * _CH, _CH)],
            neigh_hbm.at[pl.ds(base, _CH)], sems_o[b]).wait()


@functools.partial(jax.jit, static_argnums=())
def _sc_gather(nodes_p, nidx, emb):
    mesh = plsc.VectorSubcoreMesh(core_axis_name="c", subcore_axis_name="s")
    f = pl.kernel(
        _sc_body,
        out_type=(
            jax.ShapeDtypeStruct((_BP, _D), jnp.float32),
            jax.ShapeDtypeStruct((_BP, _D), jnp.float32),
        ),
        mesh=mesh,
        compiler_params=pltpu.CompilerParams(use_tc_tiling_on_sc=False),
        scratch_types=[
            pltpu.VMEM((_BW,), jnp.int32),
            pltpu.VMEM((_BW, _K), jnp.int32),
            pltpu.VMEM((_BW * _K,), jnp.int32),
            pltpu.VMEM((_CH * _K, _D), jnp.float32),
            pltpu.VMEM((_CH * _K, _D), jnp.float32),
            pltpu.VMEM_SHARED((_NS * 2 * _CH, _D), jnp.float32),
            pltpu.VMEM((_CH * _K,), jnp.int32),
            pltpu.VMEM((_CH * _K,), jnp.int32),
            pltpu.VMEM((_CH, _D), jnp.float32),
            pltpu.VMEM((80, _D), jnp.float32),
            pltpu.SemaphoreType.DMA,
            pltpu.SemaphoreType.DMA,
            pltpu.SemaphoreType.DMA,
            pltpu.SemaphoreType.DMA,
            pltpu.SemaphoreType.DMA,
            pltpu.SemaphoreType.DMA,
        ],
    )
    return f(nodes_p, nidx, emb)


def _tc_body(w1_ref, w2_ref, xs_ref, xn_ref, o_ref):
    a = lax.dot_general(w1_ref[...], xs_ref[...],
                        (((1,), (1,)), ((), ())),
                        preferred_element_type=jnp.float32)
    b = lax.dot_general(w2_ref[...], xn_ref[...],
                        (((1,), (1,)), ((), ())),
                        preferred_element_type=jnp.float32)
    o_ref[...] = jnp.maximum(a + b, 0.0)


def _tc_combine(w1, w2, xs, xn):
    blk = 2048
    grid = _BP // blk
    return pl.pallas_call(
        _tc_body,
        grid=(grid,),
        in_specs=[
            pl.BlockSpec((_D, _D), lambda i: (0, 0)),
            pl.BlockSpec((_D, _D), lambda i: (0, 0)),
            pl.BlockSpec((blk, _D), lambda i: (i, 0)),
            pl.BlockSpec((blk, _D), lambda i: (i, 0)),
        ],
        out_specs=pl.BlockSpec((_D, blk), lambda i: (0, i)),
        out_shape=jax.ShapeDtypeStruct((_D, _BP), jnp.float32),
    )(w1, w2, xs, xn)


def kernel(nodes, emb, neigh_idx, W):
    B = nodes.shape[0]
    nodes32 = nodes.astype(jnp.int32)
    nidx32 = neigh_idx.astype(jnp.int32)
    nodes_p = jnp.zeros((_BP,), jnp.int32).at[:B].set(nodes32)
    self_f, neigh_s = _sc_gather(nodes_p, nidx32, emb)
    w1 = W[:, :_D]
    w2 = W[:, _D:] * (1.0 / _K)
    out_p = _tc_combine(w1, w2, self_f, neigh_s)
    return out_p[:, :B]
